# baseline (speedup 1.0000x reference)
import functools

import jax
import jax.numpy as jnp
from jax import lax
from jax.experimental import pallas as pl
from jax.experimental.pallas import tpu as pltpu
from jax.experimental.pallas import tpu_sc as plsc

NC = 2
NS = 16
NW = NC * NS
EMB_D = 128

N_NODES = 10000
NPAD = 10240
NPT = NPAD // NS
NCHUNK = 64
NFULL = NPT // NCHUNK

N_EDGES = 320000
ECHUNK = 112
GPW = 15
CPW = GPW * 6
EPAD = NW * CPW * ECHUNK
NBODY = 7


def _sc_body(xp, es, emb, zer, comb_out, h_out,
             nidx, erows_a, erows_b, erows_c,
             sidx_0, didx_0, sidx_1, didx_1,
             comb_sh,
             asem, bsem, isem_0, isem_1,
             gsem_a, gsem_b, gsem_c, ssem_a, ssem_b, ssem_c):
    c = lax.axis_index("c")
    s = lax.axis_index("s")
    w = c * NS + s

    @pl.when(c == 1)
    def _():
        pltpu.sync_copy(zer, erows_a.at[pl.ds(0, NCHUNK)])

        def zloop(j, carry):
            nb = s * NPT + j * NCHUNK
            pltpu.sync_copy(erows_a.at[pl.ds(0, NCHUNK)],
                            comb_sh.at[pl.ds(nb, NCHUNK)])
            return carry
        lax.fori_loop(0, NFULL, zloop, 0)

    abufs = (erows_b, asem), (erows_c, bsem)

    def arows(buf):
        return buf.at[pl.ds(0, NCHUNK)]

    pltpu.sync_copy(xp.at[pl.ds(s * NPT, NCHUNK)], nidx)
    pltpu.async_copy(emb.at[nidx], arows(erows_b), asem)
    for j in range(NFULL):
        rows, sem = abufs[j % 2]
        pltpu.make_async_copy(emb.at[nidx], arows(rows), sem).wait()
        if j + 1 < NFULL:
            nrows, nsem = abufs[(j + 1) % 2]
            pltpu.sync_copy(xp.at[pl.ds(s * NPT + (j + 1) * NCHUNK, NCHUNK)],
                            nidx)
            pltpu.async_copy(emb.at[nidx], arows(nrows), nsem)
        nb = s * NPT + j * NCHUNK
        pltpu.sync_copy(arows(rows), h_out.at[pl.ds(nb, NCHUNK)])

        @pl.when(c == 0)
        def _():
            pltpu.sync_copy(arows(rows), comb_sh.at[pl.ds(nb, NCHUNK)])

    plsc.subcore_barrier()

    gb = w * GPW
    ebufs = ((erows_a, gsem_a, ssem_a),
             (erows_b, gsem_b, ssem_b),
             (erows_c, gsem_c, ssem_c))
    ibufs = ((sidx_0, didx_0, isem_0), (sidx_1, didx_1, isem_1))

    def load_grp(iset, g):
        sidx, didx, isem = ibufs[iset]
        pltpu.async_copy(es.at[0, pl.ds(g * 8, 8)], sidx, isem)
        pltpu.async_copy(es.at[1, pl.ds(g * 8, 8)], didx, isem)

    def wait_grp(iset, g):
        sidx, didx, isem = ibufs[iset]
        pltpu.make_async_copy(es.at[0, pl.ds(g * 8, 8)], sidx, isem).wait()
        pltpu.make_async_copy(es.at[1, pl.ds(g * 8, 8)], didx, isem).wait()

    def start_gather(iset, r, eb):
        erows, gsem, _ = ebufs[eb]
        pltpu.async_copy(h_out.at[ibufs[iset][0].at[r]], erows, gsem)

    def wait_gather(eb):
        erows, gsem, _ = ebufs[eb]
        pltpu.make_async_copy(h_out.at[sidx_0.at[0]], erows, gsem).wait()

    def start_scatter(iset, r, eb):
        erows, _, ssem = ebufs[eb]
        pltpu.async_copy(erows, comb_sh.at[ibufs[iset][1].at[r]], ssem,
                         add=True)

    def wait_scatter(eb):
        erows, _, ssem = ebufs[eb]
        pltpu.make_async_copy(erows, comb_sh.at[didx_0.at[0]], ssem).wait()

    load_grp(0, gb)

    def grp_body(k, carry):
        for j in range(12):
            eb = j % 3
            iset, row = (0, j) if j < 6 else (1, j - 6)
            if j >= 3:
                wait_scatter(eb)
            else:
                @pl.when(k > 0)
                def _():
                    wait_scatter(eb)
            if j == 0:
                wait_grp(0, gb + 2 * k)
            elif j == 2:
                load_grp(1, gb + 2 * k + 1)
            elif j == 6:
                wait_grp(1, gb + 2 * k + 1)
            elif j == 8:
                load_grp(0, gb + 2 * k + 2)
            start_gather(iset, row, eb)
            pb = (j - 1) % 3
            if j >= 1:
                wait_gather(pb)
                piset, prow = (0, j - 1) if j - 1 < 6 else (1, j - 7)
                start_scatter(piset, prow, pb)
            else:
                @pl.when(k > 0)
                def _():
                    wait_gather(pb)
                    start_scatter(1, 5, pb)
        return carry

    lax.fori_loop(0, NBODY, grp_body, 0)
    for j in range(6):
        eb = j % 3
        wait_scatter(eb)
        if j == 0:
            wait_grp(0, gb + 2 * NBODY)
        start_gather(0, j, eb)
        pb = (j - 1) % 3
        wait_gather(pb)
        if j == 0:
            start_scatter(1, 5, pb)
        else:
            start_scatter(0, j - 1, pb)
    wait_gather(2)
    start_scatter(0, 5, 2)
    wait_scatter(0)
    wait_scatter(1)
    wait_scatter(2)
    plsc.subcore_barrier()

    pltpu.sync_copy(comb_sh.at[pl.ds(s * NPT, NPT)],
                    comb_out.at[c, pl.ds(s * NPT, NPT)])


_sc_gnn = functools.partial(
    pl.kernel,
    out_type=(
        jax.ShapeDtypeStruct((NC, NPAD, EMB_D), jnp.float32),
        jax.ShapeDtypeStruct((NPAD, EMB_D), jnp.float32),
    ),
    mesh=plsc.VectorSubcoreMesh(
        core_axis_name="c", subcore_axis_name="s",
        num_cores=NC, num_subcores=NS),
    scratch_types=[
        pltpu.VMEM((NCHUNK,), jnp.int32),
        pltpu.VMEM((ECHUNK, EMB_D), jnp.float32),
        pltpu.VMEM((ECHUNK, EMB_D), jnp.float32),
        pltpu.VMEM((ECHUNK, EMB_D), jnp.float32),
        pltpu.VMEM((8, ECHUNK), jnp.int32),
        pltpu.VMEM((8, ECHUNK), jnp.int32),
        pltpu.VMEM((8, ECHUNK), jnp.int32),
        pltpu.VMEM((8, ECHUNK), jnp.int32),
        pltpu.VMEM_SHARED((NPAD, EMB_D), jnp.float32),
        pltpu.SemaphoreType.DMA,
        pltpu.SemaphoreType.DMA,
        pltpu.SemaphoreType.DMA,
        pltpu.SemaphoreType.DMA,
        pltpu.SemaphoreType.DMA,
        pltpu.SemaphoreType.DMA,
        pltpu.SemaphoreType.DMA,
        pltpu.SemaphoreType.DMA,
        pltpu.SemaphoreType.DMA,
        pltpu.SemaphoreType.DMA,
    ],
)(_sc_body)


BN = 2048
NBLK = NPAD // BN


def _tc_body(comb_ref, wmp_ref, bmp_ref, wcls_ref, bcls_ref, out_ref, acc_ref):
    i = pl.program_id(0)

    @pl.when(i == 0)
    def _():
        acc_ref[...] = jnp.zeros_like(acc_ref)

    cb = comb_ref[...]
    zin = cb[0] + cb[1]
    z = jax.lax.dot(zin, wmp_ref[...],
                    precision=jax.lax.Precision.HIGHEST,
                    preferred_element_type=jnp.float32)
    z = jnp.maximum(z + bmp_ref[...], 0.0)
    rid = i * BN + lax.broadcasted_iota(jnp.int32, (BN, 1), 0)
    z = jnp.where(rid < N_NODES, z, 0.0)
    acc_ref[...] += jnp.sum(z, axis=0, keepdims=True)

    @pl.when(i == NBLK - 1)
    def _():
        hg = acc_ref[...] * (1.0 / N_NODES)
        out_ref[...] = jax.lax.dot(
            hg, wcls_ref[...],
            precision=jax.lax.Precision.HIGHEST,
            preferred_element_type=jnp.float32) + bcls_ref[...]


def _tc_tail(comb, W_mp, b_mp, W_cls, b_cls):
    return pl.pallas_call(
        _tc_body,
        grid=(NBLK,),
        in_specs=[
            pl.BlockSpec((NC, BN, EMB_D), lambda i: (0, i, 0)),
            pl.BlockSpec((128, 128), lambda i: (0, 0)),
            pl.BlockSpec((1, 128), lambda i: (0, 0)),
            pl.BlockSpec((128, 16), lambda i: (0, 0)),
            pl.BlockSpec((1, 16), lambda i: (0, 0)),
        ],
        out_specs=pl.BlockSpec((1, 16), lambda i: (0, 0)),
        out_shape=jax.ShapeDtypeStruct((1, 16), jnp.float32),
        scratch_shapes=[pltpu.VMEM((1, 128), jnp.float32)],
    )(comb, W_mp, b_mp, W_cls, b_cls)


def kernel(x, edge_index, emb, W_mp, b_mp, W_cls, b_cls):
    x = x.astype(jnp.int32)
    pad = jnp.arange(NPAD - N_NODES, dtype=jnp.int32)
    xp = jnp.concatenate([x, pad])
    npe = EPAD - N_EDGES
    pe = jnp.arange(npe, dtype=jnp.int32)
    pads = jnp.stack([pe % N_NODES, N_NODES + pe % (NPAD - N_NODES)])
    es = jnp.concatenate([edge_index, pads], axis=1)
    es = es.reshape(2, NW * GPW, 6, ECHUNK)
    es = jnp.pad(es, ((0, 0), (0, 0), (0, 2), (0, 0)))
    es = es.reshape(2, NW * GPW * 8, ECHUNK)
    zer = jnp.zeros((NCHUNK, EMB_D), dtype=jnp.float32)
    comb, _h = _sc_gnn(xp, es, emb, zer)
    return _tc_tail(comb, W_mp, b_mp.reshape(1, 128), W_cls,
                    b_cls.reshape(1, 16))
